# baseline (speedup 1.0000x reference)
import functools

import jax
import jax.numpy as jnp
from jax import lax
from jax.experimental import pallas as pl
from jax.experimental.pallas import tpu as pltpu
from jax.experimental.pallas import tpu_sc as plsc

LANES = 16
CHUNK = 128


def _mm_body(x_ref, w_ref, o_ref):
    o_ref[...] = jnp.dot(x_ref[...], w_ref[...],
                         preferred_element_type=jnp.float32)


def _matmul(x, w):
    n, d_in = x.shape
    d_out = w.shape[1]
    bm = 1000
    return pl.pallas_call(
        _mm_body,
        grid=(n // bm,),
        in_specs=[
            pl.BlockSpec((bm, d_in), lambda i: (i, 0)),
            pl.BlockSpec((d_in, d_out), lambda i: (0, 0)),
        ],
        out_specs=pl.BlockSpec((bm, d_out), lambda i: (i, 0)),
        out_shape=jax.ShapeDtypeStruct((n, d_out), jnp.float32),
    )(x, w)


def _add_body(a_ref, b_ref, o_ref):
    o_ref[...] = a_ref[...] + b_ref[...]


def _add_relu_body(a_ref, b_ref, o_ref):
    o_ref[...] = jnp.maximum(a_ref[...] + b_ref[...], 0.0)


def _combine(p0, p1, relu):
    n, d = p0.shape
    bm = 1000
    return pl.pallas_call(
        _add_relu_body if relu else _add_body,
        grid=(n // bm,),
        in_specs=[
            pl.BlockSpec((bm, d), lambda i: (i, 0)),
            pl.BlockSpec((bm, d), lambda i: (i, 0)),
        ],
        out_specs=pl.BlockSpec((bm, d), lambda i: (i, 0)),
        out_shape=jax.ShapeDtypeStruct((n, d), jnp.float32),
    )(p0, p1)


@functools.cache
def _make_hop(n, d, e_pad):
    info = plsc.get_sparse_core_info()
    nc, ns = info.num_cores, info.num_subcores
    nw = nc * ns
    epw = e_pad // nw
    nchunks = epw // CHUNK
    rstride = (n // ns) // 8 * 8
    tail = n - ns * rstride
    zrows = 128
    assert epw % CHUNK == 0 and 0 <= tail < 128 and tail % 8 == 0

    mesh = plsc.VectorSubcoreMesh(core_axis_name="c", subcore_axis_name="s")

    @functools.partial(
        pl.kernel,
        mesh=mesh,
        out_type=jax.ShapeDtypeStruct((nc, n, d), jnp.float32),
        scratch_types=[
            pltpu.VMEM_SHARED((n, d), jnp.float32),
            pltpu.VMEM((CHUNK, d), jnp.float32),
            pltpu.VMEM((2 * CHUNK,), jnp.int32),
            pltpu.VMEM((CHUNK,), jnp.int32),
            pltpu.VMEM((CHUNK,), jnp.float32),
            pltpu.VMEM((128, 128), jnp.float32),
            pltpu.SemaphoreType.DMA,
        ],
    )
    def hop(h_hbm, idx_hbm, vals_hbm, out_hbm,
            acc, gat, idxv, rowv, valv, zbuf, sem):
        cid = lax.axis_index("c")
        sid = lax.axis_index("s")
        wid = sid * nc + cid

        zero16 = jnp.zeros((LANES,), jnp.float32)

        def zb(i, carry):
            for c8 in range(d // LANES):
                zbuf[i, pl.ds(c8 * LANES, LANES)] = zero16
            return carry

        lax.fori_loop(0, zrows, zb, 0)
        z0 = sid * rstride
        nfull = rstride // zrows
        rem = rstride - nfull * zrows
        for k in range(nfull):
            pltpu.sync_copy(zbuf, acc.at[pl.ds(z0 + k * zrows, zrows)])
        if rem:
            pltpu.sync_copy(zbuf.at[pl.ds(0, rem)],
                            acc.at[pl.ds(z0 + nfull * zrows, rem)])
        if tail:
            @pl.when(sid == 0)
            def _():
                pltpu.sync_copy(zbuf.at[pl.ds(0, tail)],
                                acc.at[pl.ds(ns * rstride, tail)])
        plsc.subcore_barrier()

        base = wid * epw

        def chunk_body(ci, carry):
            off = base + ci * CHUNK
            ioff = 2 * off
            pltpu.sync_copy(idx_hbm.at[pl.ds(ioff, 2 * CHUNK)], idxv)
            pltpu.sync_copy(vals_hbm.at[pl.ds(off, CHUNK)], valv)
            for t in range(CHUNK // LANES):
                sl = pl.ds(t * LANES, LANES)
                rowv[sl] = idxv[pl.ds(CHUNK + t * LANES, LANES)]
            pltpu.async_copy(h_hbm.at[idxv.at[pl.ds(0, CHUNK)]], gat,
                             sem).wait()

            def scale(j16, c2):
                vv = valv[pl.ds(j16 * LANES, LANES)]
                for i in range(LANES):
                    v = vv[i]
                    j = j16 * LANES + i
                    for c8 in range(d // LANES):
                        sl = pl.ds(c8 * LANES, LANES)
                        gat[j, sl] = gat[j, sl] * v
                return c2

            lax.fori_loop(0, CHUNK // LANES, scale, 0)
            pltpu.sync_copy(gat, acc.at[rowv], add=True)
            return carry

        lax.fori_loop(0, nchunks, chunk_body, 0)
        plsc.subcore_barrier()

        for k in range(nfull):
            pltpu.sync_copy(acc.at[pl.ds(z0 + k * zrows, zrows)],
                            out_hbm.at[cid].at[pl.ds(z0 + k * zrows, zrows)])
        if rem:
            pltpu.sync_copy(acc.at[pl.ds(z0 + nfull * zrows, rem)],
                            out_hbm.at[cid].at[pl.ds(z0 + nfull * zrows, rem)])
        if tail:
            @pl.when(sid == 0)
            def _():
                pltpu.sync_copy(acc.at[pl.ds(ns * rstride, tail)],
                                out_hbm.at[cid].at[pl.ds(ns * rstride, tail)])

    return hop


def kernel(x, edge_index, edge_vals, W):
    n, d = x.shape
    e = edge_vals.shape[0]
    rows = edge_index[0].astype(jnp.int32)
    cols = edge_index[1].astype(jnp.int32)
    vals = edge_vals.astype(jnp.float32)

    grain = 32 * CHUNK * 6
    e_pad = ((e + grain - 1) // grain) * grain
    if e_pad != e:
        pad = e_pad - e
        rows = jnp.concatenate([rows, jnp.zeros((pad,), jnp.int32)])
        cols = jnp.concatenate([cols, jnp.zeros((pad,), jnp.int32)])
        vals = jnp.concatenate([vals, jnp.zeros((pad,), jnp.float32)])

    packed = (jnp.stack([cols, rows])
              .reshape(2, e_pad // CHUNK, CHUNK)
              .swapaxes(0, 1)
              .reshape(-1))

    hop = _make_hop(n, d, e_pad)
    h = _matmul(x, W)
    p = hop(h, packed, vals)
    h = _combine(p[0], p[1], relu=False)
    p = hop(h, packed, vals)
    return _combine(p[0], p[1], relu=True)


# speedup vs baseline: 3.1341x; 3.1341x over previous
import functools

import jax
import jax.numpy as jnp
from jax import lax
from jax.experimental import pallas as pl
from jax.experimental.pallas import tpu as pltpu
from jax.experimental.pallas import tpu_sc as plsc

LANES = 16
CHUNK = 128


def _mm_body(x_ref, w_ref, o_ref):
    o_ref[...] = jnp.dot(x_ref[...], w_ref[...],
                         preferred_element_type=jnp.float32)


def _matmul(x, w):
    n, d_in = x.shape
    d_out = w.shape[1]
    bm = 1000
    return pl.pallas_call(
        _mm_body,
        grid=(n // bm,),
        in_specs=[
            pl.BlockSpec((bm, d_in), lambda i: (i, 0)),
            pl.BlockSpec((d_in, d_out), lambda i: (0, 0)),
        ],
        out_specs=pl.BlockSpec((bm, d_out), lambda i: (i, 0)),
        out_shape=jax.ShapeDtypeStruct((n, d_out), jnp.float32),
    )(x, w)


def _add_body(a_ref, b_ref, o_ref):
    o_ref[...] = a_ref[...] + b_ref[...]


def _add_relu_body(a_ref, b_ref, o_ref):
    o_ref[...] = jnp.maximum(a_ref[...] + b_ref[...], 0.0)


def _combine(p0, p1, relu):
    n, d = p0.shape
    bm = 1000
    return pl.pallas_call(
        _add_relu_body if relu else _add_body,
        grid=(n // bm,),
        in_specs=[
            pl.BlockSpec((bm, d), lambda i: (i, 0)),
            pl.BlockSpec((bm, d), lambda i: (i, 0)),
        ],
        out_specs=pl.BlockSpec((bm, d), lambda i: (i, 0)),
        out_shape=jax.ShapeDtypeStruct((n, d), jnp.float32),
    )(p0, p1)


@functools.cache
def _make_hop(n, d, e_pad):
    info = plsc.get_sparse_core_info()
    nc, ns = info.num_cores, info.num_subcores
    nw = nc * ns
    epw = e_pad // nw
    nchunks = epw // CHUNK
    rstride = (n // ns) // 8 * 8
    tail = n - ns * rstride
    zrows = 128
    assert epw % CHUNK == 0 and 0 <= tail < 128 and tail % 8 == 0

    mesh = plsc.VectorSubcoreMesh(core_axis_name="c", subcore_axis_name="s")

    @functools.partial(
        pl.kernel,
        mesh=mesh,
        out_type=jax.ShapeDtypeStruct((nc, n, d), jnp.float32),
        scratch_types=[
            pltpu.VMEM_SHARED((n, d), jnp.float32),
            pltpu.VMEM((CHUNK, d), jnp.float32),
            pltpu.VMEM((2 * CHUNK,), jnp.int32),
            pltpu.VMEM((CHUNK,), jnp.int32),
            pltpu.VMEM((CHUNK,), jnp.float32),
            pltpu.VMEM((128, 128), jnp.float32),
            pltpu.SemaphoreType.DMA,
        ],
    )
    def hop(h_hbm, idx_hbm, vals_hbm, out_hbm,
            acc, gat, idxv, rowv, valv, zbuf, sem):
        cid = lax.axis_index("c")
        sid = lax.axis_index("s")
        wid = sid * nc + cid

        zero16 = jnp.zeros((LANES,), jnp.float32)

        def zb(i, carry):
            for c8 in range(d // LANES):
                zbuf[i, pl.ds(c8 * LANES, LANES)] = zero16
            return carry

        lax.fori_loop(0, zrows, zb, 0)
        z0 = sid * rstride
        nfull = rstride // zrows
        rem = rstride - nfull * zrows
        for k in range(nfull):
            pltpu.sync_copy(zbuf, acc.at[pl.ds(z0 + k * zrows, zrows)])
        if rem:
            pltpu.sync_copy(zbuf.at[pl.ds(0, rem)],
                            acc.at[pl.ds(z0 + nfull * zrows, rem)])
        if tail:
            @pl.when(sid == 0)
            def _():
                pltpu.sync_copy(zbuf.at[pl.ds(0, tail)],
                                acc.at[pl.ds(ns * rstride, tail)])
        plsc.subcore_barrier()

        base = wid * epw

        def chunk_body(ci, carry):
            off = base + ci * CHUNK
            ioff = 2 * off
            pltpu.sync_copy(idx_hbm.at[pl.ds(ioff, 2 * CHUNK)], idxv)
            pltpu.sync_copy(vals_hbm.at[pl.ds(off, CHUNK)], valv)
            for t in range(CHUNK // LANES):
                sl = pl.ds(t * LANES, LANES)
                rowv[sl] = idxv[pl.ds(CHUNK + t * LANES, LANES)]
            pltpu.async_copy(h_hbm.at[idxv.at[pl.ds(0, CHUNK)]], gat,
                             sem).wait()

            def scale(j16, c2):
                vv = valv[pl.ds(j16 * LANES, LANES)]
                for i in range(LANES):
                    v = vv[i]
                    j = j16 * LANES + i
                    for c8 in range(d // LANES):
                        sl = pl.ds(c8 * LANES, LANES)
                        gat[j, sl] = gat[j, sl] * v
                return c2

            lax.fori_loop(0, CHUNK // LANES, scale, 0)
            pltpu.sync_copy(gat, acc.at[rowv], add=True)
            return carry

        lax.fori_loop(0, nchunks, chunk_body, 0)
        plsc.subcore_barrier()

        for k in range(nfull):
            pltpu.sync_copy(acc.at[pl.ds(z0 + k * zrows, zrows)],
                            out_hbm.at[cid].at[pl.ds(z0 + k * zrows, zrows)])
        if rem:
            pltpu.sync_copy(acc.at[pl.ds(z0 + nfull * zrows, rem)],
                            out_hbm.at[cid].at[pl.ds(z0 + nfull * zrows, rem)])
        if tail:
            @pl.when(sid == 0)
            def _():
                pltpu.sync_copy(acc.at[pl.ds(ns * rstride, tail)],
                                out_hbm.at[cid].at[pl.ds(ns * rstride, tail)])

    return hop


def kernel(x, edge_index, edge_vals, W):
    n, d = x.shape
    e = edge_vals.shape[0]
    rows = edge_index[0].astype(jnp.int32)
    cols = edge_index[1].astype(jnp.int32)
    vals = edge_vals.astype(jnp.float32)

    grain = 32 * CHUNK
    e_pad = ((e + grain - 1) // grain) * grain
    if e_pad != e:
        pad = e_pad - e
        spread = (jnp.arange(pad, dtype=jnp.int32) * 37) % n
        rows = jnp.concatenate([rows, spread])
        cols = jnp.concatenate([cols, spread])
        vals = jnp.concatenate([vals, jnp.zeros((pad,), jnp.float32)])

    packed = (jnp.stack([cols, rows])
              .reshape(2, e_pad // CHUNK, CHUNK)
              .swapaxes(0, 1)
              .reshape(-1))

    hop = _make_hop(n, d, e_pad)
    h = _matmul(x, W)
    p = hop(h, packed, vals)
    h = _combine(p[0], p[1], relu=False)
    p = hop(h, packed, vals)
    return _combine(p[0], p[1], relu=True)


# speedup vs baseline: 5.8802x; 1.8762x over previous
import functools

import jax
import jax.numpy as jnp
from jax import lax
from jax.experimental import pallas as pl
from jax.experimental.pallas import tpu as pltpu
from jax.experimental.pallas import tpu_sc as plsc

LANES = 16
CHUNK = 128


def _mm_body(x_ref, w_ref, o_ref):
    o_ref[...] = jnp.dot(x_ref[...], w_ref[...],
                         preferred_element_type=jnp.float32)


def _matmul(x, w):
    n, d_in = x.shape
    d_out = w.shape[1]
    bm = 1000
    return pl.pallas_call(
        _mm_body,
        grid=(n // bm,),
        in_specs=[
            pl.BlockSpec((bm, d_in), lambda i: (i, 0)),
            pl.BlockSpec((d_in, d_out), lambda i: (0, 0)),
        ],
        out_specs=pl.BlockSpec((bm, d_out), lambda i: (i, 0)),
        out_shape=jax.ShapeDtypeStruct((n, d_out), jnp.float32),
    )(x, w)


def _add_body(a_ref, b_ref, o_ref):
    o_ref[...] = a_ref[...] + b_ref[...]


def _add_relu_body(a_ref, b_ref, o_ref):
    o_ref[...] = jnp.maximum(a_ref[...] + b_ref[...], 0.0)


def _combine(p0, p1, relu):
    n, d = p0.shape
    bm = 1000
    return pl.pallas_call(
        _add_relu_body if relu else _add_body,
        grid=(n // bm,),
        in_specs=[
            pl.BlockSpec((bm, d), lambda i: (i, 0)),
            pl.BlockSpec((bm, d), lambda i: (i, 0)),
        ],
        out_specs=pl.BlockSpec((bm, d), lambda i: (i, 0)),
        out_shape=jax.ShapeDtypeStruct((n, d), jnp.float32),
    )(p0, p1)


G = 3
R = 2
U = 6


@functools.cache
def _make_hop(n, d, e_pad):
    info = plsc.get_sparse_core_info()
    nc, ns = info.num_cores, info.num_subcores
    nw = nc * ns
    epw = e_pad // nw
    nchunks = epw // CHUNK
    rstride = (n // ns) // 8 * 8
    tail = n - ns * rstride
    assert epw % CHUNK == 0 and nchunks % U == 0
    assert 0 <= tail < 128 and tail % 8 == 0

    mesh = plsc.VectorSubcoreMesh(core_axis_name="c", subcore_axis_name="s")

    @functools.partial(
        pl.kernel,
        mesh=mesh,
        out_type=jax.ShapeDtypeStruct((nc, n, d), jnp.float32),
        scratch_types=[
            pltpu.VMEM_SHARED((n, d), jnp.float32),
            pltpu.VMEM((G, CHUNK, d), jnp.float32),
            pltpu.VMEM((R, 2 * CHUNK), jnp.int32),
            pltpu.VMEM((R, CHUNK), jnp.float32),
            pltpu.VMEM((G, CHUNK), jnp.int32),
            pltpu.SemaphoreType.DMA,
            pltpu.SemaphoreType.DMA,
            pltpu.SemaphoreType.DMA,
            pltpu.SemaphoreType.DMA,
            pltpu.SemaphoreType.DMA,
            pltpu.SemaphoreType.DMA,
            pltpu.SemaphoreType.DMA,
            pltpu.SemaphoreType.DMA,
            pltpu.SemaphoreType.DMA,
            pltpu.SemaphoreType.DMA,
        ],
    )
    def hop(h_hbm, idx_hbm, vals_hbm, out_hbm,
            acc, gat, idxb, valr, rowr,
            sg0, sg1, sg2, ss0, ss1, ss2, si0, si1, sv0, sv1):
        sem_g = (sg0, sg1, sg2)
        sem_s = (ss0, ss1, ss2)
        sem_i = (si0, si1)
        sem_v = (sv0, sv1)
        cid = lax.axis_index("c")
        sid = lax.axis_index("s")
        wid = sid * nc + cid
        cbase = wid * nchunks

        def idx_load(k, islot):
            kc = jnp.minimum(k, nchunks - 1)
            pltpu.async_copy(idx_hbm.at[pl.ds((cbase + kc) * 2 * CHUNK,
                                              2 * CHUNK)],
                             idxb.at[islot], sem_i[islot])
            pltpu.async_copy(vals_hbm.at[pl.ds((cbase + kc) * CHUNK, CHUNK)],
                             valr.at[islot], sem_v[islot])

        def wait_i(islot):
            pltpu.make_async_copy(idx_hbm.at[pl.ds(0, 2 * CHUNK)],
                                  idxb.at[islot], sem_i[islot]).wait()
            pltpu.make_async_copy(vals_hbm.at[pl.ds(0, CHUNK)],
                                  valr.at[islot], sem_v[islot]).wait()

        def gather(islot, gslot):
            pltpu.async_copy(h_hbm.at[idxb.at[islot].at[pl.ds(0, CHUNK)]],
                             gat.at[gslot], sem_g[gslot])

        def wait_g(islot, gslot):
            pltpu.make_async_copy(h_hbm.at[idxb.at[islot].at[pl.ds(0, CHUNK)]],
                                  gat.at[gslot], sem_g[gslot]).wait()

        def scatter(gslot):
            pltpu.async_copy(gat.at[gslot], acc.at[rowr.at[gslot]],
                             sem_s[gslot], add=True)

        def wait_s(gslot):
            pltpu.make_async_copy(gat.at[gslot], acc.at[rowr.at[gslot]],
                                  sem_s[gslot]).wait()

        def rows_copy(islot, gslot):
            for t in range(CHUNK // LANES):
                rowr[gslot, pl.ds(t * LANES, LANES)] = (
                    idxb[islot, pl.ds(CHUNK + t * LANES, LANES)])

        def scale(islot, gslot):
            def body(j16, carry):
                vv = valr[islot, pl.ds(j16 * LANES, LANES)]
                for i in range(LANES):
                    v = vv[i]
                    j = j16 * LANES + i
                    for c8 in range(d // LANES):
                        sl = pl.ds(c8 * LANES, LANES)
                        gat[gslot, j, sl] = gat[gslot, j, sl] * v
                return carry

            lax.fori_loop(0, CHUNK // LANES, body, 0)

        zero16 = jnp.zeros((LANES,), jnp.float32)
        for gslot in range(G):
            def zb(i, carry, _gslot=gslot):
                for c8 in range(d // LANES):
                    gat[_gslot, i, pl.ds(c8 * LANES, LANES)] = zero16
                return carry

            lax.fori_loop(0, CHUNK, zb, 0)
        z0 = sid * rstride
        nfull = rstride // CHUNK
        rem = rstride - nfull * CHUNK
        for k in range(nfull):
            pltpu.sync_copy(gat.at[0], acc.at[pl.ds(z0 + k * CHUNK, CHUNK)])
        if rem:
            pltpu.sync_copy(gat.at[0].at[pl.ds(0, rem)],
                            acc.at[pl.ds(z0 + nfull * CHUNK, rem)])
        if tail:
            @pl.when(sid == 0)
            def _():
                pltpu.sync_copy(gat.at[0].at[pl.ds(0, tail)],
                                acc.at[pl.ds(ns * rstride, tail)])
        plsc.subcore_barrier()

        idx_load(0, 0)
        idx_load(1, 1)
        wait_i(0)
        gather(0, 0)
        rows_copy(0, 1)
        rows_copy(0, 2)
        scatter(1)
        scatter(2)

        def group(g, carry):
            k0 = g * U
            for u in range(U):
                k = k0 + u
                gslot, islot = u % G, u % R
                ngslot, nislot = (u + 1) % G, (u + 1) % R
                wait_s(ngslot)
                wait_i(nislot)
                gather(nislot, ngslot)
                wait_g(islot, gslot)
                scale(islot, gslot)
                rows_copy(islot, gslot)
                scatter(gslot)
                idx_load(k + 2, islot)
            return carry

        lax.fori_loop(0, nchunks // U, group, 0)

        wait_s(1)
        wait_s(2)
        wait_g(0, 0)
        wait_i(1)
        plsc.subcore_barrier()

        for k in range(nfull):
            pltpu.sync_copy(acc.at[pl.ds(z0 + k * CHUNK, CHUNK)],
                            out_hbm.at[cid].at[pl.ds(z0 + k * CHUNK, CHUNK)])
        if rem:
            pltpu.sync_copy(acc.at[pl.ds(z0 + nfull * CHUNK, rem)],
                            out_hbm.at[cid].at[pl.ds(z0 + nfull * CHUNK, rem)])
        if tail:
            @pl.when(sid == 0)
            def _():
                pltpu.sync_copy(acc.at[pl.ds(ns * rstride, tail)],
                                out_hbm.at[cid].at[pl.ds(ns * rstride, tail)])

    return hop


def kernel(x, edge_index, edge_vals, W):
    n, d = x.shape
    e = edge_vals.shape[0]
    rows = edge_index[0].astype(jnp.int32)
    cols = edge_index[1].astype(jnp.int32)
    vals = edge_vals.astype(jnp.float32)

    grain = 32 * CHUNK * U
    e_pad = ((e + grain - 1) // grain) * grain
    if e_pad != e:
        pad = e_pad - e
        spread = (jnp.arange(pad, dtype=jnp.int32) * 37) % n
        rows = jnp.concatenate([rows, spread])
        cols = jnp.concatenate([cols, spread])
        vals = jnp.concatenate([vals, jnp.zeros((pad,), jnp.float32)])

    packed = (jnp.stack([cols, rows])
              .reshape(2, e_pad // CHUNK, CHUNK)
              .swapaxes(0, 1)
              .reshape(-1))
    valsp = vals

    hop = _make_hop(n, d, e_pad)
    h = _matmul(x, W)
    p = hop(h, packed, valsp)
    h = _combine(p[0], p[1], relu=False)
    p = hop(h, packed, valsp)
    return _combine(p[0], p[1], relu=True)


# speedup vs baseline: 5.9988x; 1.0202x over previous
import functools

import jax
import jax.numpy as jnp
from jax import lax
from jax.experimental import pallas as pl
from jax.experimental.pallas import tpu as pltpu
from jax.experimental.pallas import tpu_sc as plsc

LANES = 16
CHUNK = 64
G = 5
R = 5
U = 5
INFLIGHT = 2


def _mm_body(x_ref, w_ref, o_ref):
    o_ref[...] = jnp.dot(x_ref[...], w_ref[...],
                         preferred_element_type=jnp.float32)


def _matmul(x, w):
    n, d_in = x.shape
    d_out = w.shape[1]
    bm = 1000
    return pl.pallas_call(
        _mm_body,
        grid=(n // bm,),
        in_specs=[
            pl.BlockSpec((bm, d_in), lambda i: (i, 0)),
            pl.BlockSpec((d_in, d_out), lambda i: (0, 0)),
        ],
        out_specs=pl.BlockSpec((bm, d_out), lambda i: (i, 0)),
        out_shape=jax.ShapeDtypeStruct((n, d_out), jnp.float32),
    )(x, w)


def _add_body(a_ref, b_ref, o_ref):
    o_ref[...] = a_ref[...] + b_ref[...]


def _add_relu_body(a_ref, b_ref, o_ref):
    o_ref[...] = jnp.maximum(a_ref[...] + b_ref[...], 0.0)


def _combine(p0, p1, relu):
    n, d = p0.shape
    bm = 1000
    return pl.pallas_call(
        _add_relu_body if relu else _add_body,
        grid=(n // bm,),
        in_specs=[
            pl.BlockSpec((bm, d), lambda i: (i, 0)),
            pl.BlockSpec((bm, d), lambda i: (i, 0)),
        ],
        out_specs=pl.BlockSpec((bm, d), lambda i: (i, 0)),
        out_shape=jax.ShapeDtypeStruct((n, d), jnp.float32),
    )(p0, p1)


@functools.cache
def _make_hop(n, d, e_pad):
    info = plsc.get_sparse_core_info()
    nc, ns = info.num_cores, info.num_subcores
    nw = nc * ns
    epw = e_pad // nw
    nchunks = epw // CHUNK
    rstride = (n // ns) // 8 * 8
    tail = n - ns * rstride
    nfull = rstride // CHUNK
    rem = rstride - nfull * CHUNK
    assert epw % CHUNK == 0 and nchunks % U == 0
    assert 0 <= tail <= CHUNK and tail % 8 == 0 and rem % 8 == 0

    mesh = plsc.VectorSubcoreMesh(core_axis_name="c", subcore_axis_name="s")

    @functools.partial(
        pl.kernel,
        mesh=mesh,
        out_type=jax.ShapeDtypeStruct((nc, n, d), jnp.float32),
        scratch_types=[
            pltpu.VMEM_SHARED((n, d), jnp.float32),
            pltpu.VMEM((G, CHUNK, d), jnp.float32),
            pltpu.VMEM((R, 2 * CHUNK), jnp.int32),
            pltpu.VMEM((R, CHUNK), jnp.float32),
            pltpu.VMEM((G, CHUNK), jnp.int32),
        ] + [pltpu.SemaphoreType.DMA] * (2 * G + 2 * R),
    )
    def hop(h_hbm, idx_hbm, vals_hbm, out_hbm,
            acc, gat, idxb, valr, rowr, *sems):
        sem_g = sems[:G]
        sem_s = sems[G:2 * G]
        sem_i = sems[2 * G:2 * G + R]
        sem_v = sems[2 * G + R:]
        cid = lax.axis_index("c")
        sid = lax.axis_index("s")
        wid = sid * nc + cid
        cbase = wid * nchunks

        def idx_load(k, islot):
            kc = jnp.minimum(k, nchunks - 1)
            pltpu.async_copy(idx_hbm.at[pl.ds((cbase + kc) * 2 * CHUNK,
                                              2 * CHUNK)],
                             idxb.at[islot], sem_i[islot])
            pltpu.async_copy(vals_hbm.at[pl.ds((cbase + kc) * CHUNK, CHUNK)],
                             valr.at[islot], sem_v[islot])

        def wait_i(islot):
            pltpu.make_async_copy(idx_hbm.at[pl.ds(0, 2 * CHUNK)],
                                  idxb.at[islot], sem_i[islot]).wait()
            pltpu.make_async_copy(vals_hbm.at[pl.ds(0, CHUNK)],
                                  valr.at[islot], sem_v[islot]).wait()

        def gather(islot, gslot):
            pltpu.async_copy(h_hbm.at[idxb.at[islot].at[pl.ds(0, CHUNK)]],
                             gat.at[gslot], sem_g[gslot])

        def wait_g(islot, gslot):
            pltpu.make_async_copy(h_hbm.at[idxb.at[islot].at[pl.ds(0, CHUNK)]],
                                  gat.at[gslot], sem_g[gslot]).wait()

        def scatter(gslot):
            pltpu.async_copy(gat.at[gslot], acc.at[rowr.at[gslot]],
                             sem_s[gslot], add=True)

        def wait_s(gslot):
            pltpu.make_async_copy(gat.at[gslot], acc.at[rowr.at[gslot]],
                                  sem_s[gslot]).wait()

        def rows_copy(islot, gslot):
            for t in range(CHUNK // LANES):
                rowr[gslot, pl.ds(t * LANES, LANES)] = (
                    idxb[islot, pl.ds(CHUNK + t * LANES, LANES)])

        def scale(islot, gslot):
            def body(j16, carry):
                vv = valr[islot, pl.ds(j16 * LANES, LANES)]
                for i in range(LANES):
                    v = vv[i]
                    j = j16 * LANES + i
                    for c8 in range(d // LANES):
                        sl = pl.ds(c8 * LANES, LANES)
                        gat[gslot, j, sl] = gat[gslot, j, sl] * v
                return carry

            lax.fori_loop(0, CHUNK // LANES, body, 0)

        zero16 = jnp.zeros((LANES,), jnp.float32)
        for gslot in range(G):
            def zb(i, carry, _gslot=gslot):
                for c8 in range(d // LANES):
                    gat[_gslot, i, pl.ds(c8 * LANES, LANES)] = zero16
                return carry

            lax.fori_loop(0, CHUNK, zb, 0)
        z0 = sid * rstride
        for k in range(nfull):
            pltpu.sync_copy(gat.at[0], acc.at[pl.ds(z0 + k * CHUNK, CHUNK)])
        if rem:
            pltpu.sync_copy(gat.at[0].at[pl.ds(0, rem)],
                            acc.at[pl.ds(z0 + nfull * CHUNK, rem)])
        if tail:
            @pl.when(sid == 0)
            def _():
                pltpu.sync_copy(gat.at[0].at[pl.ds(0, tail)],
                                acc.at[pl.ds(ns * rstride, tail)])
        plsc.subcore_barrier()

        for k in range(R):
            idx_load(k, k)
        for k in range(INFLIGHT):
            wait_i(k)
            gather(k, k)
        for gslot in range(INFLIGHT, G):
            rows_copy(0, gslot)
            scatter(gslot)

        def group(g, carry):
            k0 = g * U
            for u in range(U):
                k = k0 + u
                gslot, islot = u % G, u % R
                g2, i2 = (u + INFLIGHT) % G, (u + INFLIGHT) % R
                wait_s(g2)
                wait_i(i2)
                gather(i2, g2)
                wait_g(islot, gslot)
                scale(islot, gslot)
                rows_copy(islot, gslot)
                scatter(gslot)
                idx_load(k + R, islot)
            return carry

        lax.fori_loop(0, nchunks // U, group, 0)

        for gslot in range(INFLIGHT, G):
            wait_s(gslot)
        for k in range(INFLIGHT):
            wait_g(k, k)
        for s_ in range(INFLIGHT, R):
            wait_i(s_)
        plsc.subcore_barrier()

        for k in range(nfull):
            pltpu.sync_copy(acc.at[pl.ds(z0 + k * CHUNK, CHUNK)],
                            out_hbm.at[cid].at[pl.ds(z0 + k * CHUNK, CHUNK)])
        if rem:
            pltpu.sync_copy(acc.at[pl.ds(z0 + nfull * CHUNK, rem)],
                            out_hbm.at[cid].at[pl.ds(z0 + nfull * CHUNK, rem)])
        if tail:
            @pl.when(sid == 0)
            def _():
                pltpu.sync_copy(acc.at[pl.ds(ns * rstride, tail)],
                                out_hbm.at[cid].at[pl.ds(ns * rstride, tail)])

    return hop


def kernel(x, edge_index, edge_vals, W):
    n, d = x.shape
    e = edge_vals.shape[0]
    rows = edge_index[0].astype(jnp.int32)
    cols = edge_index[1].astype(jnp.int32)
    vals = edge_vals.astype(jnp.float32)

    grain = 32 * CHUNK * U
    e_pad = ((e + grain - 1) // grain) * grain
    if e_pad != e:
        pad = e_pad - e
        spread = (jnp.arange(pad, dtype=jnp.int32) * 37) % n
        rows = jnp.concatenate([rows, spread])
        cols = jnp.concatenate([cols, spread])
        vals = jnp.concatenate([vals, jnp.zeros((pad,), jnp.float32)])

    packed = (jnp.stack([cols, rows])
              .reshape(2, e_pad // CHUNK, CHUNK)
              .swapaxes(0, 1)
              .reshape(-1))

    hop = _make_hop(n, d, e_pad)
    h = _matmul(x, W)
    p = hop(h, packed, vals)
    h = _combine(p[0], p[1], relu=False)
    p = hop(h, packed, vals)
    return _combine(p[0], p[1], relu=True)


# speedup vs baseline: 6.0890x; 1.0150x over previous
import functools

import jax
import jax.numpy as jnp
from jax import lax
from jax.experimental import pallas as pl
from jax.experimental.pallas import tpu as pltpu
from jax.experimental.pallas import tpu_sc as plsc

LANES = 16
CHUNK = 64
G = 5
R = 5
U = 5
INFLIGHT = 3


def _mm_body(x_ref, w_ref, o_ref):
    o_ref[...] = jnp.dot(x_ref[...], w_ref[...],
                         preferred_element_type=jnp.float32)


def _matmul(x, w):
    n, d_in = x.shape
    d_out = w.shape[1]
    bm = 1000
    return pl.pallas_call(
        _mm_body,
        grid=(n // bm,),
        in_specs=[
            pl.BlockSpec((bm, d_in), lambda i: (i, 0)),
            pl.BlockSpec((d_in, d_out), lambda i: (0, 0)),
        ],
        out_specs=pl.BlockSpec((bm, d_out), lambda i: (i, 0)),
        out_shape=jax.ShapeDtypeStruct((n, d_out), jnp.float32),
    )(x, w)


def _add_body(a_ref, b_ref, o_ref):
    o_ref[...] = a_ref[...] + b_ref[...]


def _add_relu_body(a_ref, b_ref, o_ref):
    o_ref[...] = jnp.maximum(a_ref[...] + b_ref[...], 0.0)


def _combine(p0, p1, relu):
    n, d = p0.shape
    bm = 1000
    return pl.pallas_call(
        _add_relu_body if relu else _add_body,
        grid=(n // bm,),
        in_specs=[
            pl.BlockSpec((bm, d), lambda i: (i, 0)),
            pl.BlockSpec((bm, d), lambda i: (i, 0)),
        ],
        out_specs=pl.BlockSpec((bm, d), lambda i: (i, 0)),
        out_shape=jax.ShapeDtypeStruct((n, d), jnp.float32),
    )(p0, p1)


@functools.cache
def _make_hop(n, d, e_pad):
    info = plsc.get_sparse_core_info()
    nc, ns = info.num_cores, info.num_subcores
    nw = nc * ns
    epw = e_pad // nw
    nchunks = epw // CHUNK
    rstride = (n // ns) // 8 * 8
    tail = n - ns * rstride
    nfull = rstride // CHUNK
    rem = rstride - nfull * CHUNK
    assert epw % CHUNK == 0 and nchunks % U == 0
    assert 0 <= tail <= CHUNK and tail % 8 == 0 and rem % 8 == 0

    mesh = plsc.VectorSubcoreMesh(core_axis_name="c", subcore_axis_name="s")

    @functools.partial(
        pl.kernel,
        mesh=mesh,
        out_type=jax.ShapeDtypeStruct((nc, n, d), jnp.float32),
        scratch_types=[
            pltpu.VMEM_SHARED((n, d), jnp.float32),
            pltpu.VMEM((G, CHUNK, d), jnp.float32),
            pltpu.VMEM((R, 2 * CHUNK), jnp.int32),
            pltpu.VMEM((R, CHUNK), jnp.float32),
            pltpu.VMEM((G, CHUNK), jnp.int32),
        ] + [pltpu.SemaphoreType.DMA] * (2 * G + 2 * R),
    )
    def hop(h_hbm, idx_hbm, vals_hbm, out_hbm,
            acc, gat, idxb, valr, rowr, *sems):
        sem_g = sems[:G]
        sem_s = sems[G:2 * G]
        sem_i = sems[2 * G:2 * G + R]
        sem_v = sems[2 * G + R:]
        cid = lax.axis_index("c")
        sid = lax.axis_index("s")
        wid = sid * nc + cid
        cbase = wid * nchunks

        def idx_load(k, islot):
            kc = jnp.minimum(k, nchunks - 1)
            pltpu.async_copy(idx_hbm.at[pl.ds((cbase + kc) * 2 * CHUNK,
                                              2 * CHUNK)],
                             idxb.at[islot], sem_i[islot])
            pltpu.async_copy(vals_hbm.at[pl.ds((cbase + kc) * CHUNK, CHUNK)],
                             valr.at[islot], sem_v[islot])

        def wait_i(islot):
            pltpu.make_async_copy(idx_hbm.at[pl.ds(0, 2 * CHUNK)],
                                  idxb.at[islot], sem_i[islot]).wait()
            pltpu.make_async_copy(vals_hbm.at[pl.ds(0, CHUNK)],
                                  valr.at[islot], sem_v[islot]).wait()

        def gather(islot, gslot):
            pltpu.async_copy(h_hbm.at[idxb.at[islot].at[pl.ds(0, CHUNK)]],
                             gat.at[gslot], sem_g[gslot])

        def wait_g(islot, gslot):
            pltpu.make_async_copy(h_hbm.at[idxb.at[islot].at[pl.ds(0, CHUNK)]],
                                  gat.at[gslot], sem_g[gslot]).wait()

        def scatter(gslot):
            pltpu.async_copy(gat.at[gslot], acc.at[rowr.at[gslot]],
                             sem_s[gslot], add=True)

        def wait_s(gslot):
            pltpu.make_async_copy(gat.at[gslot], acc.at[rowr.at[gslot]],
                                  sem_s[gslot]).wait()

        def rows_copy(islot, gslot):
            for t in range(CHUNK // LANES):
                rowr[gslot, pl.ds(t * LANES, LANES)] = (
                    idxb[islot, pl.ds(CHUNK + t * LANES, LANES)])

        def scale(islot, gslot):
            def body(j16, carry):
                vv = valr[islot, pl.ds(j16 * LANES, LANES)]
                for i in range(LANES):
                    v = vv[i]
                    j = j16 * LANES + i
                    for c8 in range(d // LANES):
                        sl = pl.ds(c8 * LANES, LANES)
                        gat[gslot, j, sl] = gat[gslot, j, sl] * v
                return carry

            lax.fori_loop(0, CHUNK // LANES, body, 0)

        zero16 = jnp.zeros((LANES,), jnp.float32)
        for gslot in range(G):
            def zb(i, carry, _gslot=gslot):
                for c8 in range(d // LANES):
                    gat[_gslot, i, pl.ds(c8 * LANES, LANES)] = zero16
                return carry

            lax.fori_loop(0, CHUNK, zb, 0)
        z0 = sid * rstride
        for k in range(nfull):
            pltpu.sync_copy(gat.at[0], acc.at[pl.ds(z0 + k * CHUNK, CHUNK)])
        if rem:
            pltpu.sync_copy(gat.at[0].at[pl.ds(0, rem)],
                            acc.at[pl.ds(z0 + nfull * CHUNK, rem)])
        if tail:
            @pl.when(sid == 0)
            def _():
                pltpu.sync_copy(gat.at[0].at[pl.ds(0, tail)],
                                acc.at[pl.ds(ns * rstride, tail)])
        plsc.subcore_barrier()

        for k in range(R):
            idx_load(k, k)
        for k in range(INFLIGHT):
            wait_i(k)
            gather(k, k)
        for gslot in range(INFLIGHT, G):
            rows_copy(0, gslot)
            scatter(gslot)

        def group(g, carry):
            k0 = g * U
            for u in range(U):
                k = k0 + u
                gslot, islot = u % G, u % R
                g2, i2 = (u + INFLIGHT) % G, (u + INFLIGHT) % R
                wait_s(g2)
                wait_i(i2)
                gather(i2, g2)
                wait_g(islot, gslot)
                scale(islot, gslot)
                rows_copy(islot, gslot)
                scatter(gslot)
                idx_load(k + R, islot)
            return carry

        lax.fori_loop(0, nchunks // U, group, 0)

        for gslot in range(INFLIGHT, G):
            wait_s(gslot)
        for k in range(INFLIGHT):
            wait_g(k, k)
        for s_ in range(INFLIGHT, R):
            wait_i(s_)
        plsc.subcore_barrier()

        for k in range(nfull):
            pltpu.sync_copy(acc.at[pl.ds(z0 + k * CHUNK, CHUNK)],
                            out_hbm.at[cid].at[pl.ds(z0 + k * CHUNK, CHUNK)])
        if rem:
            pltpu.sync_copy(acc.at[pl.ds(z0 + nfull * CHUNK, rem)],
                            out_hbm.at[cid].at[pl.ds(z0 + nfull * CHUNK, rem)])
        if tail:
            @pl.when(sid == 0)
            def _():
                pltpu.sync_copy(acc.at[pl.ds(ns * rstride, tail)],
                                out_hbm.at[cid].at[pl.ds(ns * rstride, tail)])

    return hop


def kernel(x, edge_index, edge_vals, W):
    n, d = x.shape
    e = edge_vals.shape[0]
    rows = edge_index[0].astype(jnp.int32)
    cols = edge_index[1].astype(jnp.int32)
    vals = edge_vals.astype(jnp.float32)

    grain = 32 * CHUNK * U
    e_pad = ((e + grain - 1) // grain) * grain
    if e_pad != e:
        pad = e_pad - e
        spread = (jnp.arange(pad, dtype=jnp.int32) * 37) % n
        rows = jnp.concatenate([rows, spread])
        cols = jnp.concatenate([cols, spread])
        vals = jnp.concatenate([vals, jnp.zeros((pad,), jnp.float32)])

    packed = (jnp.stack([cols, rows])
              .reshape(2, e_pad // CHUNK, CHUNK)
              .swapaxes(0, 1)
              .reshape(-1))

    hop = _make_hop(n, d, e_pad)
    h = _matmul(x, W)
    p = hop(h, packed, vals)
    h = _combine(p[0], p[1], relu=False)
    p = hop(h, packed, vals)
    return _combine(p[0], p[1], relu=True)
